# Initial kernel scaffold; baseline (speedup 1.0000x reference)
import jax
import jax.numpy as jnp
from jax.experimental import pallas as pl


def kernel(white_indices, black_indices, W_ft, W1, b1, W2, b2, W3, b3, W_out, b_out):
    raise NotImplementedError("write your pallas kernel here")


# speedup vs baseline: 7.8850x; 7.8850x over previous
import functools

import jax
import jax.numpy as jnp
from jax import lax
from jax.experimental import pallas as pl
from jax.experimental.pallas import tpu as pltpu
from jax.experimental.pallas import tpu_sc as plsc

_L = 16


def _make_seg_sum(S, A, F, H):
    NC, NS = 2, 16
    NW = NC * NS
    assert S % NW == 0
    SEG_W = S // NW
    KH = H // _L

    mesh = plsc.VectorSubcoreMesh(core_axis_name="c", subcore_axis_name="s")

    @functools.partial(
        pl.kernel,
        mesh=mesh,
        out_type=jax.ShapeDtypeStruct((S, H), jnp.float32),
        scratch_types=[
            pltpu.VMEM((SEG_W, A), jnp.int32),
            pltpu.VMEM((2, A, H), jnp.float32),
            pltpu.VMEM((SEG_W, H), jnp.float32),
            pltpu.VMEM((2 * _L,), jnp.float32),
            pltpu.VMEM((2 * _L,), jnp.int32),
            pltpu.SemaphoreType.DMA,
            pltpu.SemaphoreType.DMA,
        ],
        compiler_params=pltpu.CompilerParams(needs_layout_passes=False),
    )
    def seg_sum(idx_hbm, table_hbm, out_hbm, idx_v, rows_v, out_v, w_v, idx_s,
                sem0, sem1):
        wid = lax.axis_index("s") * NC + lax.axis_index("c")
        base = wid * SEG_W
        pltpu.sync_copy(idx_hbm.at[pl.ds(base, SEG_W)], idx_v)

        def gather_desc(j, buf, sem):
            return pltpu.make_async_copy(
                table_hbm.at[idx_v.at[j]], rows_v.at[buf], sem)

        gather_desc(0, 0, sem0).start()
        gather_desc(1, 1, sem1).start()

        lane = lax.iota(jnp.int32, _L)

        def compute_seg(j, buf, sem):
            gather_desc(j, buf, sem).wait()
            u = idx_v[j, pl.ds(0, _L)]
            v = idx_v[j, pl.ds(_L, _L)]
            idx_s[pl.ds(0, _L)] = u
            idx_s[pl.ds(_L, _L)] = v
            cnt_u = jnp.ones((_L,), jnp.int32)
            cnt_v = jnp.ones((_L,), jnp.int32)
            for s in range(1, _L):
                rot = (lane + s) & (_L - 1)
                cnt_u += (u == plsc.load_gather(idx_s, [rot])).astype(jnp.int32)
                cnt_v += (v == plsc.load_gather(idx_s, [rot + _L])).astype(jnp.int32)
            for s in range(_L):
                rot = (lane + s) & (_L - 1)
                cnt_u += (u == plsc.load_gather(idx_s, [rot + _L])).astype(jnp.int32)
                cnt_v += (v == plsc.load_gather(idx_s, [rot])).astype(jnp.int32)
            w_v[pl.ds(0, _L)] = 1.0 / cnt_u.astype(jnp.float32)
            w_v[pl.ds(_L, _L)] = 1.0 / cnt_v.astype(jnp.float32)

            def acc_body(a, acc):
                wb = plsc.load_gather(w_v, [jnp.full((_L,), 0, jnp.int32) + a])
                return tuple(
                    acc[k] + wb * rows_v[buf, a, pl.ds(k * _L, _L)]
                    for k in range(KH))

            acc = lax.fori_loop(
                0, A, acc_body,
                tuple(jnp.zeros((_L,), jnp.float32) for _ in range(KH)))
            for k in range(KH):
                out_v[j, pl.ds(k * _L, _L)] = acc[k]

            @pl.when(j + 2 < SEG_W)
            def _():
                gather_desc(j + 2, buf, sem).start()

        def body2(i, carry):
            compute_seg(2 * i, 0, sem0)
            compute_seg(2 * i + 1, 1, sem1)
            return carry

        lax.fori_loop(0, SEG_W // 2, body2, 0)
        pltpu.sync_copy(out_v, out_hbm.at[pl.ds(base, SEG_W)])

    return seg_sum


def _mlp_body(xw_ref, xb_ref, w1_ref, b1_ref, w2_ref, b2_ref, w3_ref, b3_ref,
              wo_ref, bo_ref, o_ref):
    dn = (((1,), (1,)), ((), ()))
    xw = jnp.clip(xw_ref[...], -1.0, 1.0)
    xb = jnp.clip(xb_ref[...], -1.0, 1.0)
    h = jnp.concatenate([xw, xb], axis=1)
    h = jnp.maximum(
        lax.dot_general(h, w1_ref[...], dn, preferred_element_type=jnp.float32)
        + b1_ref[...], 0.0)
    h = jnp.maximum(
        lax.dot_general(h, w2_ref[...], dn, preferred_element_type=jnp.float32)
        + b2_ref[...], 0.0)
    h = jnp.maximum(
        lax.dot_general(h, w3_ref[...], dn, preferred_element_type=jnp.float32)
        + b3_ref[...], 0.0)
    o_ref[...] = jnp.sum(h * wo_ref[...], axis=1, keepdims=True) + bo_ref[...]


def _mlp(acc, W1, b1, W2, b2, W3, b3, W_out, b_out):
    S, H = acc.shape
    B = S // 2
    BB = 256
    NB = B // BB
    H2, H4 = W2.shape[0], W3.shape[0]
    full = lambda shape: pl.BlockSpec(shape, lambda i: (0, 0))
    return pl.pallas_call(
        _mlp_body,
        grid=(NB,),
        in_specs=[
            pl.BlockSpec((BB, H), lambda i: (i, 0)),
            pl.BlockSpec((BB, H), lambda i: (i + NB, 0)),
            full((H, 2 * H)),
            full((1, H)),
            full((H2, H)),
            full((1, H2)),
            full((H4, H2)),
            full((1, H4)),
            full((1, H4)),
            full((1, 1)),
        ],
        out_specs=pl.BlockSpec((BB, 1), lambda i: (i, 0)),
        out_shape=jax.ShapeDtypeStruct((B, 1), jnp.float32),
    )(acc, acc, W1, b1.reshape(1, -1), W2, b2.reshape(1, -1),
      W3, b3.reshape(1, -1), W_out, b_out.reshape(1, 1))


def kernel(white_indices, black_indices, W_ft, W1, b1, W2, b2, W3, b3, W_out, b_out):
    B, A = white_indices.shape
    H, F = W_ft.shape
    idx_all = jnp.concatenate([white_indices, black_indices], axis=0)
    table = W_ft.T
    seg_sum = _make_seg_sum(2 * B, A, F, H)
    acc = seg_sum(idx_all, table)
    out = _mlp(acc, W1, b1, W2, b2, W3, b3, W_out, b_out)
    return out[:, 0]
